# baseline (speedup 1.0000x reference)
import functools

import jax
import jax.numpy as jnp
from jax.experimental import pallas as pl

_LAYERS = [(32, 4, 32, True), (128, 4, 32, True), (128, 1, 32, False)]


def _mm_body(x_ref, w_ref, o_ref):
    o_ref[...] = jnp.dot(x_ref[...], w_ref[...], preferred_element_type=jnp.float32)


def _mm(x, w, bn):
    n, k = x.shape
    m = w.shape[1]
    assert n % bn == 0, (n, bn)
    return pl.pallas_call(
        _mm_body,
        grid=(n // bn,),
        in_specs=[
            pl.BlockSpec((bn, k), lambda i: (i, 0)),
            pl.BlockSpec((k, m), lambda i: (0, 0)),
        ],
        out_specs=pl.BlockSpec((bn, m), lambda i: (i, 0)),
        out_shape=jax.ShapeDtypeStruct((n, m), jnp.float32),
    )(x, w)


def _segment_softmax(alpha, seg, num_segments):
    m = jax.ops.segment_max(alpha, seg, num_segments)
    m = jnp.where(jnp.isfinite(m), m, 0.0)
    ex = jnp.exp(alpha - m[seg])
    denom = jax.ops.segment_sum(ex, seg, num_segments)
    return ex / (denom[seg] + 1e-16)


def _gat_layer(x, src, dst, a_edge, p, heads, out_ch, concat):
    num_nodes = x.shape[0]
    h = _mm(x, p['W'], 2000).reshape(num_nodes, heads, out_ch)
    a_src = (h * p['att_src']).sum(-1)
    a_dst = (h * p['att_dst']).sum(-1)
    alpha = a_src[src] + a_dst[dst] + a_edge
    alpha = jax.nn.leaky_relu(alpha, 0.2)
    alpha = _segment_softmax(alpha, dst, num_nodes)
    msg = h[src] * alpha[..., None]
    out = jax.ops.segment_sum(msg, dst, num_nodes)
    if concat:
        out = out.reshape(num_nodes, heads * out_ch)
    else:
        out = out.mean(axis=1)
    return out + p['bias']


def kernel(x, edge_index, edge_attr, params):
    src = edge_index[0]
    dst = edge_index[1]
    we_atts = []
    for p in params:
        h_, c_ = p['att_src'].shape[1], p['att_src'].shape[2]
        we = p['W_e'].reshape(p['W_e'].shape[0], h_, c_)
        we_atts.append((we * p['att_edge']).sum(-1))
    we_all = jnp.concatenate(we_atts, axis=1)
    we_all = jnp.pad(we_all, ((0, 0), (0, 16 - we_all.shape[1])))
    a_edge_all = _mm(edge_attr, we_all, 16000)

    h = x
    off = 0
    n_layers = len(params)
    for i, p in enumerate(params):
        heads, out_ch = p['att_src'].shape[1], p['att_src'].shape[2]
        concat = i < n_layers - 1 or p['bias'].shape[0] == heads * out_ch
        a_edge = a_edge_all[:, off:off + heads]
        off += heads
        h = _gat_layer(h, src, dst, a_edge, p, heads, out_ch,
                       concat=(p['bias'].shape[0] == heads * out_ch))
        if i < n_layers - 1:
            h = jax.nn.elu(h)
    return h
